# baseline (speedup 1.0000x reference)
import jax
import jax.numpy as jnp
from jax import lax
from jax.experimental import pallas as pl
from jax.experimental.pallas import tpu as pltpu
from jax.experimental.pallas import tpu_sc as plsc

N = 10000
E = 320000
F_IN = 128
HID = 16
C = 40

NC = 2
NS = 16
W = NC * NS
EPW = E // W
CH = 128
NCHF = 78
TAIL = EPW - NCHF * CH
NB = 13
NP = 10240
RP = N // NS
DP = NP // NS
GB = 10
RB = N // GB


def _mesh():
    return plsc.VectorSubcoreMesh(core_axis_name="c", subcore_axis_name="s")


_SC_PARAMS = pltpu.CompilerParams(
    use_tc_tiling_on_sc=False, needs_layout_passes=False)


def _rsqrt_sc(d):
    bi = plsc.bitcast(d, jnp.int32)
    yi = jnp.int32(0x5F3759DF) - lax.shift_right_logical(bi, 1)
    y = plsc.bitcast(yi, jnp.float32)
    y = y * (1.5 - 0.5 * d * y * y)
    y = y * (1.5 - 0.5 * d * y * y)
    return y


def _agg_loop(sp_table, src_flat, dst_flat, rows, acc, gsem, ssem):
    def aggloop(g, cy):
        gd = [
            pltpu.async_copy(
                sp_table.at[src_flat.at[pl.ds((g * NB + b) * CH, CH)]],
                rows.at[b], gsem)
            for b in range(NB)
        ]
        sd = []
        for b in range(NB):
            gd[b].wait()
            sd.append(
                pltpu.async_copy(
                    rows.at[b],
                    acc.at[dst_flat.at[pl.ds((g * NB + b) * CH, CH)]],
                    ssem, add=True)
            )
        for d in sd:
            d.wait()
        return cy

    lax.fori_loop(0, NCHF // NB, aggloop, 0)
    pltpu.async_copy(
        sp_table.at[src_flat.at[pl.ds(NCHF * CH, TAIL)]],
        rows.at[0, pl.ds(0, TAIL)], gsem).wait()
    pltpu.async_copy(
        rows.at[0, pl.ds(0, TAIL)],
        acc.at[dst_flat.at[pl.ds(NCHF * CH, TAIL)]], ssem, add=True).wait()


def _deg_body(ei, ones_h, zerosd, out, dst_flat, ones_v, dacc, ssem):
    c = lax.axis_index("c")
    s = lax.axis_index("s")
    w = c * NS + s
    d0 = s * DP

    pltpu.sync_copy(ei.at[1, pl.ds(w * EPW, EPW)], dst_flat)
    pltpu.sync_copy(ones_h, ones_v)
    pltpu.sync_copy(zerosd, dacc.at[pl.ds(d0, DP)])
    plsc.subcore_barrier()

    def degloop(g, cy):
        sd = [
            pltpu.async_copy(
                ones_v,
                dacc.at[dst_flat.at[pl.ds((g * NB + b) * CH, CH)]],
                ssem, add=True)
            for b in range(NB)
        ]
        for d in sd:
            d.wait()
        return cy

    lax.fori_loop(0, NCHF // NB, degloop, 0)
    pltpu.async_copy(
        ones_v.at[pl.ds(0, TAIL)],
        dacc.at[dst_flat.at[pl.ds(NCHF * CH, TAIL)]], ssem, add=True).wait()
    plsc.subcore_barrier()
    pltpu.sync_copy(dacc.at[pl.ds(d0, DP)], out.at[c, pl.ds(d0, DP)])


def _sc_degree(ei, ones_h, zerosd):
    k = pl.kernel(
        _deg_body,
        out_type=jax.ShapeDtypeStruct((NC, NP), jnp.float32),
        mesh=_mesh(),
        compiler_params=_SC_PARAMS,
        scratch_types=[
            pltpu.VMEM((EPW,), jnp.int32),
            pltpu.VMEM((CH,), jnp.float32),
            pltpu.VMEM_SHARED((NP,), jnp.float32),
            pltpu.SemaphoreType.DMA,
        ],
    )
    return k(ei, ones_h, zerosd)


def _mega1_body(mm, ei, degp, zeros16,
                a1, t1o, dinv16o,
                src_idx, dst_idx, rows, dp0, dp1, dv16, mv,
                t1_sp, acc, dinv_sp, gsem, ssem):
    c = lax.axis_index("c")
    s = lax.axis_index("s")
    w = c * NS + s
    r0 = s * RP
    d0 = s * DP

    pltpu.sync_copy(ei.at[0, pl.ds(w * EPW, EPW)], src_idx)
    pltpu.sync_copy(ei.at[1, pl.ds(w * EPW, EPW)], dst_idx)
    pltpu.sync_copy(zeros16, acc.at[pl.ds(d0, DP)])
    pltpu.sync_copy(degp.at[0, pl.ds(d0, DP)], dp0)
    pltpu.sync_copy(degp.at[1, pl.ds(d0, DP)], dp1)
    pltpu.sync_copy(mm.at[pl.ds(r0, RP)], mv)

    def dloop(g, cy):
        base = g * 16
        y = _rsqrt_sc(dp0[pl.ds(base, 16)] + dp1[pl.ds(base, 16)] + 1.0)
        for i in range(16):
            dv16[base + i] = jnp.full((16,), 1.0, jnp.float32) * y[i]
        return cy

    lax.fori_loop(0, DP // 16, dloop, 0)
    pltpu.sync_copy(dv16, dinv_sp.at[pl.ds(d0, DP)])
    plsc.subcore_barrier()

    pltpu.sync_copy(dinv_sp.at[pl.ds(r0, RP)], dv16.at[pl.ds(0, RP)])

    def sloop(g, cy):
        for k in range(5):
            r = g * 5 + k
            mv[r] = mv[r] * dv16[r]
        return cy

    lax.fori_loop(0, RP // 5, sloop, 0)
    pltpu.sync_copy(mv, t1_sp.at[pl.ds(r0, RP)])

    @pl.when(c == 0)
    def _():
        pltpu.sync_copy(mv, t1o.at[pl.ds(r0, RP)])
        pltpu.sync_copy(dv16.at[pl.ds(0, RP)], dinv16o.at[pl.ds(r0, RP)])

    plsc.subcore_barrier()
    _agg_loop(t1_sp, src_idx, dst_idx, rows, acc, gsem, ssem)
    plsc.subcore_barrier()
    pltpu.sync_copy(acc.at[pl.ds(r0, RP)], a1.at[c, pl.ds(r0, RP)])


def _mega1(mm, ei, degp, zeros16):
    k = pl.kernel(
        _mega1_body,
        out_type=(
            jax.ShapeDtypeStruct((NC, N, HID), jnp.float32),
            jax.ShapeDtypeStruct((N, HID), jnp.float32),
            jax.ShapeDtypeStruct((N, HID), jnp.float32),
        ),
        mesh=_mesh(),
        compiler_params=_SC_PARAMS,
        scratch_types=[
            pltpu.VMEM((EPW,), jnp.int32),
            pltpu.VMEM((EPW,), jnp.int32),
            pltpu.VMEM((NB, CH, HID), jnp.float32),
            pltpu.VMEM((DP,), jnp.float32),
            pltpu.VMEM((DP,), jnp.float32),
            pltpu.VMEM((DP, HID), jnp.float32),
            pltpu.VMEM((RP, HID), jnp.float32),
            pltpu.VMEM_SHARED((NP, HID), jnp.float32),
            pltpu.VMEM_SHARED((NP, HID), jnp.float32),
            pltpu.VMEM_SHARED((NP, HID), jnp.float32),
            pltpu.SemaphoreType.DMA,
            pltpu.SemaphoreType.DMA,
        ],
    )
    return k(mm, ei, degp, zeros16)


def _mega2_body(a1, t1o, dinv16o, b1h, ei, zeros16,
                a2, t2o,
                src_idx, dst_idx, rows, b1v, dvt, p0, p1, t1s,
                t2_sp, acc, gsem, ssem):
    c = lax.axis_index("c")
    s = lax.axis_index("s")
    w = c * NS + s
    r0 = s * RP
    d0 = s * DP

    pltpu.sync_copy(ei.at[0, pl.ds(w * EPW, EPW)], src_idx)
    pltpu.sync_copy(ei.at[1, pl.ds(w * EPW, EPW)], dst_idx)
    pltpu.sync_copy(b1h, b1v)
    pltpu.sync_copy(zeros16, acc.at[pl.ds(d0, DP)])
    pltpu.sync_copy(dinv16o.at[pl.ds(r0, RP)], dvt)
    pltpu.sync_copy(a1.at[0, pl.ds(r0, RP)], p0)
    pltpu.sync_copy(a1.at[1, pl.ds(r0, RP)], p1)
    pltpu.sync_copy(t1o.at[pl.ds(r0, RP)], t1s)
    b1vec = b1v[...]

    def tloop(g, cy):
        for k in range(5):
            r = g * 5 + k
            y = dvt[r]
            z = y * (p0[r] + p1[r] + t1s[r]) + b1vec
            p0[r] = y * jnp.maximum(z, 0.0)
        return cy

    lax.fori_loop(0, RP // 5, tloop, 0)
    pltpu.sync_copy(p0, t2_sp.at[pl.ds(r0, RP)])

    @pl.when(c == 0)
    def _():
        pltpu.sync_copy(p0, t2o.at[pl.ds(r0, RP)])

    plsc.subcore_barrier()
    _agg_loop(t2_sp, src_idx, dst_idx, rows, acc, gsem, ssem)
    plsc.subcore_barrier()
    pltpu.sync_copy(acc.at[pl.ds(r0, RP)], a2.at[c, pl.ds(r0, RP)])


def _mega2(a1, t1o, dinv16o, b1, ei, zeros16):
    k = pl.kernel(
        _mega2_body,
        out_type=(
            jax.ShapeDtypeStruct((NC, N, HID), jnp.float32),
            jax.ShapeDtypeStruct((N, HID), jnp.float32),
        ),
        mesh=_mesh(),
        compiler_params=_SC_PARAMS,
        scratch_types=[
            pltpu.VMEM((EPW,), jnp.int32),
            pltpu.VMEM((EPW,), jnp.int32),
            pltpu.VMEM((NB, CH, HID), jnp.float32),
            pltpu.VMEM((HID,), jnp.float32),
            pltpu.VMEM((RP, HID), jnp.float32),
            pltpu.VMEM((RP, HID), jnp.float32),
            pltpu.VMEM((RP, HID), jnp.float32),
            pltpu.VMEM((RP, HID), jnp.float32),
            pltpu.VMEM_SHARED((NP, HID), jnp.float32),
            pltpu.VMEM_SHARED((NP, HID), jnp.float32),
            pltpu.SemaphoreType.DMA,
            pltpu.SemaphoreType.DMA,
        ],
    )
    return k(a1, t1o, dinv16o, b1, ei, zeros16)


def _tc_mm_body(x_ref, w1_ref, out_ref):
    out_ref[...] = jnp.dot(
        x_ref[...], w1_ref[...], preferred_element_type=jnp.float32)


def _tc_mm(x, w1):
    return pl.pallas_call(
        _tc_mm_body,
        out_shape=jax.ShapeDtypeStruct((N, HID), jnp.float32),
    )(x, w1)


def _tc_c_body(a_ref, t2_ref, dinv_ref, w2_ref, b2_ref, out_ref):
    u = dinv_ref[...] * (a_ref[0] + a_ref[1] + t2_ref[...])
    z = jnp.dot(u, w2_ref[...], preferred_element_type=jnp.float32) + b2_ref[...]
    m = jnp.max(z, axis=1, keepdims=True)
    zs = z - m
    lse = jnp.log(jnp.sum(jnp.exp(zs), axis=1, keepdims=True))
    out_ref[...] = zs - lse


def _tc_c(a, t2, dinv16, w2, b2):
    return pl.pallas_call(
        _tc_c_body,
        grid=(GB,),
        in_specs=[
            pl.BlockSpec((NC, RB, HID), lambda i: (0, i, 0)),
            pl.BlockSpec((RB, HID), lambda i: (i, 0)),
            pl.BlockSpec((RB, HID), lambda i: (i, 0)),
            pl.BlockSpec((HID, C), lambda i: (0, 0)),
            pl.BlockSpec((1, C), lambda i: (0, 0)),
        ],
        out_specs=pl.BlockSpec((RB, C), lambda i: (i, 0)),
        out_shape=jax.ShapeDtypeStruct((N, C), jnp.float32),
    )(a, t2, dinv16, w2, b2)


def kernel(x, edge_index, W1, b1, W2, b2):
    ones_h = jnp.ones((CH,), jnp.float32)
    zerosd = jnp.zeros((DP,), jnp.float32)
    zeros16 = jnp.zeros((DP, HID), jnp.float32)

    degp = _sc_degree(edge_index, ones_h, zerosd)
    mm = _tc_mm(x, W1)
    a1, t1o, dinv16o = _mega1(mm, edge_index, degp, zeros16)
    a2, t2o = _mega2(a1, t1o, dinv16o, b1, edge_index, zeros16)
    return _tc_c(a2, t2o, dinv16o, W2, b2.reshape(1, C))


# speedup vs baseline: 1.0196x; 1.0196x over previous
import jax
import jax.numpy as jnp
from jax import lax
from jax.experimental import pallas as pl
from jax.experimental.pallas import tpu as pltpu
from jax.experimental.pallas import tpu_sc as plsc

N = 10000
E = 320000
F_IN = 128
HID = 16
C = 40

NC = 2
NS = 16
W = NC * NS
EPW = E // W
CH = 128
NCHF = 78
TAIL = EPW - NCHF * CH
NB = 13
NP = 10240
RP = N // NS
DP = NP // NS
GB = 5
RB = N // GB


def _mesh():
    return plsc.VectorSubcoreMesh(core_axis_name="c", subcore_axis_name="s")


_SC_PARAMS = pltpu.CompilerParams(
    use_tc_tiling_on_sc=False, needs_layout_passes=False)


def _rsqrt_sc(d):
    bi = plsc.bitcast(d, jnp.int32)
    yi = jnp.int32(0x5F3759DF) - lax.shift_right_logical(bi, 1)
    y = plsc.bitcast(yi, jnp.float32)
    y = y * (1.5 - 0.5 * d * y * y)
    y = y * (1.5 - 0.5 * d * y * y)
    return y


def _agg_loop(sp_table, src_flat, dst_flat, rows, acc, gsem, ssem):
    def aggloop(g, cy):
        gd = [
            pltpu.async_copy(
                sp_table.at[src_flat.at[pl.ds((g * NB + b) * CH, CH)]],
                rows.at[b], gsem)
            for b in range(NB)
        ]
        sd = []
        for b in range(NB):
            gd[b].wait()
            sd.append(
                pltpu.async_copy(
                    rows.at[b],
                    acc.at[dst_flat.at[pl.ds((g * NB + b) * CH, CH)]],
                    ssem, add=True)
            )
        for d in sd:
            d.wait()
        return cy

    lax.fori_loop(0, NCHF // NB, aggloop, 0)
    pltpu.async_copy(
        sp_table.at[src_flat.at[pl.ds(NCHF * CH, TAIL)]],
        rows.at[0, pl.ds(0, TAIL)], gsem).wait()
    pltpu.async_copy(
        rows.at[0, pl.ds(0, TAIL)],
        acc.at[dst_flat.at[pl.ds(NCHF * CH, TAIL)]], ssem, add=True).wait()


def _deg_body(ei, ones_h, zerosd, out, dst_flat, ones_v, dacc, ssem):
    c = lax.axis_index("c")
    s = lax.axis_index("s")
    w = c * NS + s
    d0 = s * DP

    pltpu.sync_copy(ei.at[1, pl.ds(w * EPW, EPW)], dst_flat)
    pltpu.sync_copy(ones_h, ones_v)
    pltpu.sync_copy(zerosd, dacc.at[pl.ds(d0, DP)])
    plsc.subcore_barrier()

    def degloop(g, cy):
        sd = [
            pltpu.async_copy(
                ones_v,
                dacc.at[dst_flat.at[pl.ds((g * NB + b) * CH, CH)]],
                ssem, add=True)
            for b in range(NB)
        ]
        for d in sd:
            d.wait()
        return cy

    lax.fori_loop(0, NCHF // NB, degloop, 0)
    pltpu.async_copy(
        ones_v.at[pl.ds(0, TAIL)],
        dacc.at[dst_flat.at[pl.ds(NCHF * CH, TAIL)]], ssem, add=True).wait()
    plsc.subcore_barrier()
    pltpu.sync_copy(dacc.at[pl.ds(d0, DP)], out.at[c, pl.ds(d0, DP)])


def _sc_degree(ei, ones_h, zerosd):
    k = pl.kernel(
        _deg_body,
        out_type=jax.ShapeDtypeStruct((NC, NP), jnp.float32),
        mesh=_mesh(),
        compiler_params=_SC_PARAMS,
        scratch_types=[
            pltpu.VMEM((EPW,), jnp.int32),
            pltpu.VMEM((CH,), jnp.float32),
            pltpu.VMEM_SHARED((NP,), jnp.float32),
            pltpu.SemaphoreType.DMA,
        ],
    )
    return k(ei, ones_h, zerosd)


def _mega1_body(mm, ei, degp, zeros16,
                a1, t1o, dinv16o,
                src_idx, dst_idx, rows, dp0, dp1, dv16, mv,
                t1_sp, acc, dinv_sp, gsem, ssem):
    c = lax.axis_index("c")
    s = lax.axis_index("s")
    w = c * NS + s
    r0 = s * RP
    d0 = s * DP

    pltpu.sync_copy(ei.at[0, pl.ds(w * EPW, EPW)], src_idx)
    pltpu.sync_copy(ei.at[1, pl.ds(w * EPW, EPW)], dst_idx)
    pltpu.sync_copy(zeros16, acc.at[pl.ds(d0, DP)])
    pltpu.sync_copy(degp.at[0, pl.ds(d0, DP)], dp0)
    pltpu.sync_copy(degp.at[1, pl.ds(d0, DP)], dp1)
    pltpu.sync_copy(mm.at[pl.ds(r0, RP)], mv)

    def dloop(g, cy):
        base = g * 16
        y = _rsqrt_sc(dp0[pl.ds(base, 16)] + dp1[pl.ds(base, 16)] + 1.0)
        for i in range(16):
            dv16[base + i] = jnp.full((16,), 1.0, jnp.float32) * y[i]
        return cy

    lax.fori_loop(0, DP // 16, dloop, 0)
    pltpu.sync_copy(dv16, dinv_sp.at[pl.ds(d0, DP)])
    plsc.subcore_barrier()

    pltpu.sync_copy(dinv_sp.at[pl.ds(r0, RP)], dv16.at[pl.ds(0, RP)])

    def sloop(g, cy):
        for k in range(5):
            r = g * 5 + k
            mv[r] = mv[r] * dv16[r]
        return cy

    lax.fori_loop(0, RP // 5, sloop, 0)
    pltpu.sync_copy(mv, t1_sp.at[pl.ds(r0, RP)])

    @pl.when(c == 0)
    def _():
        pltpu.sync_copy(mv, t1o.at[pl.ds(r0, RP)])
        pltpu.sync_copy(dv16.at[pl.ds(0, RP)], dinv16o.at[pl.ds(r0, RP)])

    plsc.subcore_barrier()
    _agg_loop(t1_sp, src_idx, dst_idx, rows, acc, gsem, ssem)
    plsc.subcore_barrier()
    pltpu.sync_copy(acc.at[pl.ds(r0, RP)], a1.at[c, pl.ds(r0, RP)])


def _mega1(mm, ei, degp, zeros16):
    k = pl.kernel(
        _mega1_body,
        out_type=(
            jax.ShapeDtypeStruct((NC, N, HID), jnp.float32),
            jax.ShapeDtypeStruct((N, HID), jnp.float32),
            jax.ShapeDtypeStruct((N, HID), jnp.float32),
        ),
        mesh=_mesh(),
        compiler_params=_SC_PARAMS,
        scratch_types=[
            pltpu.VMEM((EPW,), jnp.int32),
            pltpu.VMEM((EPW,), jnp.int32),
            pltpu.VMEM((NB, CH, HID), jnp.float32),
            pltpu.VMEM((DP,), jnp.float32),
            pltpu.VMEM((DP,), jnp.float32),
            pltpu.VMEM((DP, HID), jnp.float32),
            pltpu.VMEM((RP, HID), jnp.float32),
            pltpu.VMEM_SHARED((NP, HID), jnp.float32),
            pltpu.VMEM_SHARED((NP, HID), jnp.float32),
            pltpu.VMEM_SHARED((NP, HID), jnp.float32),
            pltpu.SemaphoreType.DMA,
            pltpu.SemaphoreType.DMA,
        ],
    )
    return k(mm, ei, degp, zeros16)


def _mega2_body(a1, t1o, dinv16o, b1h, ei, zeros16,
                a2, t2o,
                src_idx, dst_idx, rows, b1v, dvt, p0, p1, t1s,
                t2_sp, acc, gsem, ssem):
    c = lax.axis_index("c")
    s = lax.axis_index("s")
    w = c * NS + s
    r0 = s * RP
    d0 = s * DP

    pltpu.sync_copy(ei.at[0, pl.ds(w * EPW, EPW)], src_idx)
    pltpu.sync_copy(ei.at[1, pl.ds(w * EPW, EPW)], dst_idx)
    pltpu.sync_copy(b1h, b1v)
    pltpu.sync_copy(zeros16, acc.at[pl.ds(d0, DP)])
    pltpu.sync_copy(dinv16o.at[pl.ds(r0, RP)], dvt)
    pltpu.sync_copy(a1.at[0, pl.ds(r0, RP)], p0)
    pltpu.sync_copy(a1.at[1, pl.ds(r0, RP)], p1)
    pltpu.sync_copy(t1o.at[pl.ds(r0, RP)], t1s)
    b1vec = b1v[...]

    def tloop(g, cy):
        for k in range(5):
            r = g * 5 + k
            y = dvt[r]
            z = y * (p0[r] + p1[r] + t1s[r]) + b1vec
            p0[r] = y * jnp.maximum(z, 0.0)
        return cy

    lax.fori_loop(0, RP // 5, tloop, 0)
    pltpu.sync_copy(p0, t2_sp.at[pl.ds(r0, RP)])

    @pl.when(c == 0)
    def _():
        pltpu.sync_copy(p0, t2o.at[pl.ds(r0, RP)])

    plsc.subcore_barrier()
    _agg_loop(t2_sp, src_idx, dst_idx, rows, acc, gsem, ssem)
    plsc.subcore_barrier()
    pltpu.sync_copy(acc.at[pl.ds(r0, RP)], a2.at[c, pl.ds(r0, RP)])


def _mega2(a1, t1o, dinv16o, b1, ei, zeros16):
    k = pl.kernel(
        _mega2_body,
        out_type=(
            jax.ShapeDtypeStruct((NC, N, HID), jnp.float32),
            jax.ShapeDtypeStruct((N, HID), jnp.float32),
        ),
        mesh=_mesh(),
        compiler_params=_SC_PARAMS,
        scratch_types=[
            pltpu.VMEM((EPW,), jnp.int32),
            pltpu.VMEM((EPW,), jnp.int32),
            pltpu.VMEM((NB, CH, HID), jnp.float32),
            pltpu.VMEM((HID,), jnp.float32),
            pltpu.VMEM((RP, HID), jnp.float32),
            pltpu.VMEM((RP, HID), jnp.float32),
            pltpu.VMEM((RP, HID), jnp.float32),
            pltpu.VMEM((RP, HID), jnp.float32),
            pltpu.VMEM_SHARED((NP, HID), jnp.float32),
            pltpu.VMEM_SHARED((NP, HID), jnp.float32),
            pltpu.SemaphoreType.DMA,
            pltpu.SemaphoreType.DMA,
        ],
    )
    return k(a1, t1o, dinv16o, b1, ei, zeros16)


def _tc_mm_body(x_ref, w1_ref, out_ref):
    out_ref[...] = jnp.dot(
        x_ref[...], w1_ref[...], preferred_element_type=jnp.float32)


def _tc_mm(x, w1):
    return pl.pallas_call(
        _tc_mm_body,
        out_shape=jax.ShapeDtypeStruct((N, HID), jnp.float32),
    )(x, w1)


def _tc_c_body(a_ref, t2_ref, dinv_ref, w2_ref, b2_ref, out_ref):
    u = dinv_ref[...] * (a_ref[0] + a_ref[1] + t2_ref[...])
    z = jnp.dot(u, w2_ref[...], preferred_element_type=jnp.float32) + b2_ref[...]
    m = jnp.max(z, axis=1, keepdims=True)
    zs = z - m
    lse = jnp.log(jnp.sum(jnp.exp(zs), axis=1, keepdims=True))
    out_ref[...] = zs - lse


def _tc_c(a, t2, dinv16, w2, b2):
    return pl.pallas_call(
        _tc_c_body,
        grid=(GB,),
        in_specs=[
            pl.BlockSpec((NC, RB, HID), lambda i: (0, i, 0)),
            pl.BlockSpec((RB, HID), lambda i: (i, 0)),
            pl.BlockSpec((RB, HID), lambda i: (i, 0)),
            pl.BlockSpec((HID, C), lambda i: (0, 0)),
            pl.BlockSpec((1, C), lambda i: (0, 0)),
        ],
        out_specs=pl.BlockSpec((RB, C), lambda i: (i, 0)),
        out_shape=jax.ShapeDtypeStruct((N, C), jnp.float32),
    )(a, t2, dinv16, w2, b2)


def kernel(x, edge_index, W1, b1, W2, b2):
    ones_h = jnp.ones((CH,), jnp.float32)
    zerosd = jnp.zeros((DP,), jnp.float32)
    zeros16 = jnp.zeros((DP, HID), jnp.float32)

    degp = _sc_degree(edge_index, ones_h, zerosd)
    mm = _tc_mm(x, W1)
    a1, t1o, dinv16o = _mega1(mm, edge_index, degp, zeros16)
    a2, t2o = _mega2(a1, t1o, dinv16o, b1, edge_index, zeros16)
    return _tc_c(a2, t2o, dinv16o, W2, b2.reshape(1, C))
